# baseline (speedup 1.0000x reference)
import functools

import jax
import jax.numpy as jnp
from jax.experimental import pallas as pl
from jax.experimental.pallas import tpu as pltpu

_HIGHEST = jax.lax.Precision.HIGHEST


def _mm(a, b):
    return jax.lax.dot_general(
        a, b, (((1,), (0,)), ((), ())),
        precision=_HIGHEST, preferred_element_type=jnp.float32)


def _mmd(a, b):
    return jax.lax.dot_general(
        a, b, (((1,), (0,)), ((), ())),
        preferred_element_type=jnp.float32)


def _fps_body(px_ref, py_ref, pz_ref, cx_ref, cy_ref, cz_ref, *, ns):
    px = px_ref[...]
    py = py_ref[...]
    pz = pz_ref[...]
    B, N = px.shape
    lane = jax.lax.broadcasted_iota(jnp.int32, (B, N), 1)
    col = jax.lax.broadcasted_iota(jnp.int32, (B, ns), 1)

    cx_ref[...] = jnp.zeros((B, ns), jnp.float32)
    cy_ref[...] = jnp.zeros((B, ns), jnp.float32)
    cz_ref[...] = jnp.zeros((B, ns), jnp.float32)

    def step(t, carry):
        dists, far = carry
        selm = (lane == far).astype(jnp.float32)
        cxv = jnp.sum(selm * px, axis=1, keepdims=True)
        cyv = jnp.sum(selm * py, axis=1, keepdims=True)
        czv = jnp.sum(selm * pz, axis=1, keepdims=True)
        tm = (col == t).astype(jnp.float32)
        cx_ref[...] += tm * cxv
        cy_ref[...] += tm * cyv
        cz_ref[...] += tm * czv
        d = (px - cxv) ** 2 + (py - cyv) ** 2 + (pz - czv) ** 2
        dists = jnp.minimum(dists, d)
        mx = jnp.max(dists, axis=1, keepdims=True)
        far = jnp.min(jnp.where(dists == mx, lane, N), axis=1, keepdims=True)
        return dists, far

    dists0 = jnp.full((B, N), 1e10, jnp.float32)
    far0 = jnp.zeros((B, 1), jnp.int32)
    jax.lax.fori_loop(0, ns, step, (dists0, far0))


def _fps_call(px, py, pz, ns):
    B, N = px.shape
    body = functools.partial(_fps_body, ns=ns)
    shp = jax.ShapeDtypeStruct((B, ns), jnp.float32)
    return pl.pallas_call(body, out_shape=[shp, shp, shp])(px, py, pz)


def _knn_body(px_ref, py_ref, pz_ref, cx_ref, cy_ref, cz_ref, feats_ref,
              w_ref, b_ref, y_ref, ssum_ref, ssq_ref, d_ref, *, k,
              coord_only):
    b_id = pl.program_id(0)
    S = cx_ref.shape[2]
    N = px_ref.shape[2]
    Cout = w_ref.shape[1]

    px = px_ref[0]
    py = py_ref[0]
    pz = pz_ref[0]
    cxc = jnp.reshape(cx_ref[0], (S, 1))
    cyc = jnp.reshape(cy_ref[0], (S, 1))
    czc = jnp.reshape(cz_ref[0], (S, 1))
    d_ref[...] = (cxc - px) ** 2 + (cyc - py) ** 2 + (czc - pz) ** 2

    W = w_ref[...]
    bb = b_ref[...]
    lane = jax.lax.broadcasted_iota(jnp.int32, (S, N), 1)
    if not coord_only:
        F = feats_ref[0]
        Cin = F.shape[1]
        CC = jnp.concatenate(
            [cxc, cyc, czc, jnp.zeros((S, Cin - 3), jnp.float32)], axis=1)

    def round_fn(r, _):
        D = d_ref[...]
        mn = jnp.min(D, axis=1, keepdims=True)
        jm = jnp.min(jnp.where(D == mn, lane, N), axis=1, keepdims=True)
        M1 = lane == jm
        M1f = M1.astype(jnp.float32)
        if coord_only:
            nbx = jnp.sum(M1f * px, axis=1, keepdims=True)
            nby = jnp.sum(M1f * py, axis=1, keepdims=True)
            nbz = jnp.sum(M1f * pz, axis=1, keepdims=True)
            g = jnp.concatenate(
                [nbx - cxc, nby - cyc, nbz - czc, nbx, nby, nbz], axis=1)
        else:
            g = _mm(M1f, F) - CC
        y = _mmd(g, W) + bb
        y_ref[0, r] = y
        d_ref[...] = jnp.where(M1, 1e30, D)
        return 0

    jax.lax.fori_loop(0, k, round_fn, 0)
    yall = jnp.reshape(y_ref[0], (k * S, Cout))
    ssum_ref[0] = jnp.sum(yall, axis=0, keepdims=True)
    ssq_ref[0] = jnp.sum(yall * yall, axis=0, keepdims=True)


def _knn_call(px, py, pz, cx, cy, cz, feats, wt, bb, k):
    B, N = px.shape
    S = cx.shape[1]
    Cin, Cout = wt.shape
    px = px[:, None, :]
    py = py[:, None, :]
    pz = pz[:, None, :]
    cx = cx[:, None, :]
    cy = cy[:, None, :]
    cz = cz[:, None, :]
    coord_only = feats is None
    body = functools.partial(_knn_body, k=k, coord_only=coord_only)
    row_specs = [pl.BlockSpec((1, 1, N), lambda b: (b, 0, 0))] * 3 + [
        pl.BlockSpec((1, 1, S), lambda b: (b, 0, 0))] * 3
    if coord_only:
        feats_args = ()
        feats_specs = []
    else:
        feats_args = (feats,)
        feats_specs = [pl.BlockSpec((1, N, Cin), lambda b: (b, 0, 0))]

    def body_wrap(*refs):
        if coord_only:
            (px_r, py_r, pz_r, cx_r, cy_r, cz_r, w_r, b_r,
             y_r, s_r, q_r, d_r) = refs
            body(px_r, py_r, pz_r, cx_r, cy_r, cz_r, None, w_r, b_r,
                 y_r, s_r, q_r, d_r)
        else:
            body(*refs)

    return pl.pallas_call(
        body_wrap,
        grid=(B,),
        in_specs=row_specs + feats_specs + [
            pl.BlockSpec((Cin, Cout), lambda b: (0, 0)),
            pl.BlockSpec((1, Cout), lambda b: (0, 0)),
        ],
        out_specs=[
            pl.BlockSpec((1, k, S, Cout), lambda b: (b, 0, 0, 0)),
            pl.BlockSpec((1, 1, Cout), lambda b: (b, 0, 0)),
            pl.BlockSpec((1, 1, Cout), lambda b: (b, 0, 0)),
        ],
        out_shape=[
            jax.ShapeDtypeStruct((B, k, S, Cout), jnp.float32),
            jax.ShapeDtypeStruct((B, 1, Cout), jnp.float32),
            jax.ShapeDtypeStruct((B, 1, Cout), jnp.float32),
        ],
        scratch_shapes=[pltpu.VMEM((S, N), jnp.float32)],
    )(px, py, pz, cx, cy, cz, *feats_args, wt, bb)


def _bn_conv_max_body(y_ref, ssum_ref, ssq_ref, g_ref, be_ref, w_ref, b2_ref,
                      out_ref, *, count):
    K, S, C = y_ref.shape[1:]
    C2 = w_ref.shape[1]
    nb = ssum_ref.shape[0]
    m = jnp.sum(jnp.reshape(ssum_ref[...], (nb, C)), axis=0,
                keepdims=True) / count
    ex2 = jnp.sum(jnp.reshape(ssq_ref[...], (nb, C)), axis=0,
                  keepdims=True) / count
    v = ex2 - m * m
    den = jnp.sqrt(v + 1e-5)
    yn = (y_ref[0] - m[None]) / den[None]
    h = jnp.maximum(g_ref[...][None] * yn + be_ref[...][None], 0.0)
    o = _mmd(jnp.reshape(h, (K * S, C)), w_ref[...]) + b2_ref[...]
    out_ref[0] = jnp.max(jnp.reshape(o, (K, S, C2)), axis=0)


def _pass2_call(y, ssum, ssq, g, be, wt, b2):
    B, K, S, C = y.shape
    C2 = wt.shape[1]
    body = functools.partial(_bn_conv_max_body, count=float(B * K * S))
    return pl.pallas_call(
        body,
        grid=(B,),
        in_specs=[
            pl.BlockSpec((1, K, S, C), lambda b: (b, 0, 0, 0)),
            pl.BlockSpec((B, 1, C), lambda b: (0, 0, 0)),
            pl.BlockSpec((B, 1, C), lambda b: (0, 0, 0)),
            pl.BlockSpec((1, C), lambda b: (0, 0)),
            pl.BlockSpec((1, C), lambda b: (0, 0)),
            pl.BlockSpec((C, C2), lambda b: (0, 0)),
            pl.BlockSpec((1, C2), lambda b: (0, 0)),
        ],
        out_specs=pl.BlockSpec((1, S, C2), lambda b: (b, 0, 0)),
        out_shape=jax.ShapeDtypeStruct((B, S, C2), jnp.float32),
    )(y, ssum, ssq, g, be, wt, b2)


def _tail_body(x_ref, w3a_ref, b3a_ref, g3_ref,
               be3_ref, w3b_ref, b3b_ref, fw1_ref, fb1_ref, fg1_ref, fbe1_ref,
               fw2_ref, fb2_ref, fg2_ref, fbe2_ref, fw3_ref, fb3_ref, out_ref,
               *, nb):
    B = nb
    S = x_ref.shape[0] // nb
    y = _mmd(x_ref[...], w3a_ref[...]) + b3a_ref[...]
    m = jnp.mean(y, axis=0, keepdims=True)
    v = jnp.mean((y - m) ** 2, axis=0, keepdims=True)
    yn = (y - m) / jnp.sqrt(v + 1e-5)
    h = jnp.maximum(g3_ref[...] * yn + be3_ref[...], 0.0)
    y2 = _mmd(h, w3b_ref[...]) + b3b_ref[...]
    gf = jnp.max(jnp.reshape(y2, (B, S, -1)), axis=1)

    def fc_bn_lrelu(x, wt, bv, gv, bev):
        hh = _mmd(x, wt) + bv
        mu = jnp.mean(hh, axis=0, keepdims=True)
        vv = jnp.mean((hh - mu) ** 2, axis=0, keepdims=True)
        nn = gv * (hh - mu) / jnp.sqrt(vv + 1e-5) + bev
        return jnp.where(nn >= 0, nn, 0.2 * nn)

    h1 = fc_bn_lrelu(gf, fw1_ref[...], fb1_ref[...], fg1_ref[...],
                     fbe1_ref[...])
    h2 = fc_bn_lrelu(h1, fw2_ref[...], fb2_ref[...], fg2_ref[...],
                     fbe2_ref[...])
    out_ref[...] = _mmd(h2, fw3_ref[...]) + fb3_ref[...]


def _tail_call(cx, cy, cz, l2, *weights):
    B, S = cx.shape
    C = l2.shape[2]
    coords = jnp.stack([cx, cy, cz], axis=2).reshape(B * S, 3)
    x = jnp.concatenate([coords, l2.reshape(B * S, C)], axis=1)
    body = functools.partial(_tail_body, nb=B)
    return pl.pallas_call(
        body,
        out_shape=jax.ShapeDtypeStruct((B, 1), jnp.float32),
    )(x, *weights)


def kernel(points, w1a, b1a, g1a, be1a, w1b, b1b, w2a, b2a, g2a, be2a, w2b,
           b2b, w3a, b3a, g3a, be3a, w3b, b3b, fw1, fb1, fg1, fbe1, fw2, fb2,
           fg2, fbe2, fw3, fb3):
    points = points.astype(jnp.float32)
    px = points[:, :, 0]
    py = points[:, :, 1]
    pz = points[:, :, 2]

    c1x, c1y, c1z = _fps_call(px, py, pz, 512)
    y1, s1, q1 = _knn_call(px, py, pz, c1x, c1y, c1z, None,
                           w1a.T, b1a[None], k=16)
    l1 = _pass2_call(y1, s1, q1, g1a[None], be1a[None], w1b.T, b1b[None])

    c2x, c2y, c2z = _fps_call(c1x, c1y, c1z, 128)
    feats2 = jnp.concatenate([jnp.stack([c1x, c1y, c1z], axis=2), l1], axis=2)
    y2, s2, q2 = _knn_call(c1x, c1y, c1z, c2x, c2y, c2z, feats2,
                           w2a.T, b2a[None], k=16)
    l2 = _pass2_call(y2, s2, q2, g2a[None], be2a[None], w2b.T, b2b[None])

    return _tail_call(c2x, c2y, c2z, l2,
                      w3a.T, b3a[None], g3a[None], be3a[None],
                      w3b.T, b3b[None],
                      fw1.T, fb1[None], fg1[None], fbe1[None],
                      fw2.T, fb2[None], fg2[None], fbe2[None],
                      fw3.T, fb3[None])


# speedup vs baseline: 1.0781x; 1.0781x over previous
import functools

import jax
import jax.numpy as jnp
from jax.experimental import pallas as pl
from jax.experimental.pallas import tpu as pltpu

_HIGHEST = jax.lax.Precision.HIGHEST


def _mm(a, b):
    return jax.lax.dot_general(
        a, b, (((1,), (0,)), ((), ())),
        precision=_HIGHEST, preferred_element_type=jnp.float32)


def _mmd(a, b):
    return jax.lax.dot_general(
        a, b, (((1,), (0,)), ((), ())),
        preferred_element_type=jnp.float32)


def _fps_body(px_ref, py_ref, pz_ref, cx_ref, cy_ref, cz_ref, *, ns):
    px = px_ref[...]
    py = py_ref[...]
    pz = pz_ref[...]
    B, N = px.shape
    lane = jax.lax.broadcasted_iota(jnp.int32, (B, N), 1)
    col = jax.lax.broadcasted_iota(jnp.int32, (B, ns), 1)

    cx_ref[...] = jnp.zeros((B, ns), jnp.float32)
    cy_ref[...] = jnp.zeros((B, ns), jnp.float32)
    cz_ref[...] = jnp.zeros((B, ns), jnp.float32)

    def step(t, carry):
        dists, far = carry
        selm = (lane == far).astype(jnp.float32)
        cxv = jnp.sum(selm * px, axis=1, keepdims=True)
        cyv = jnp.sum(selm * py, axis=1, keepdims=True)
        czv = jnp.sum(selm * pz, axis=1, keepdims=True)
        tm = (col == t).astype(jnp.float32)
        cx_ref[...] += tm * cxv
        cy_ref[...] += tm * cyv
        cz_ref[...] += tm * czv
        d = (px - cxv) ** 2 + (py - cyv) ** 2 + (pz - czv) ** 2
        dists = jnp.minimum(dists, d)
        far = jnp.argmax(dists, axis=1, keepdims=True).astype(jnp.int32)
        return dists, far

    dists0 = jnp.full((B, N), 1e10, jnp.float32)
    far0 = jnp.zeros((B, 1), jnp.int32)
    jax.lax.fori_loop(0, ns, step, (dists0, far0))


def _fps_call(px, py, pz, ns):
    B, N = px.shape
    body = functools.partial(_fps_body, ns=ns)
    shp = jax.ShapeDtypeStruct((B, ns), jnp.float32)
    return pl.pallas_call(body, out_shape=[shp, shp, shp])(px, py, pz)


def _knn_body(px_ref, py_ref, pz_ref, cx_ref, cy_ref, cz_ref, feats_ref,
              w_ref, b_ref, y_ref, ssum_ref, ssq_ref, d_ref, *, k,
              coord_only):
    b_id = pl.program_id(0)
    S = cx_ref.shape[2]
    N = px_ref.shape[2]
    Cout = w_ref.shape[1]

    px = px_ref[0]
    py = py_ref[0]
    pz = pz_ref[0]
    cxc = jnp.reshape(cx_ref[0], (S, 1))
    cyc = jnp.reshape(cy_ref[0], (S, 1))
    czc = jnp.reshape(cz_ref[0], (S, 1))
    d_ref[...] = (cxc - px) ** 2 + (cyc - py) ** 2 + (czc - pz) ** 2

    W = w_ref[...]
    bb = b_ref[...]
    lane = jax.lax.broadcasted_iota(jnp.int32, (S, N), 1)
    if not coord_only:
        F = feats_ref[0]
        Cin = F.shape[1]
        CC = jnp.concatenate(
            [cxc, cyc, czc, jnp.zeros((S, Cin - 3), jnp.float32)], axis=1)

    def round_fn(r, _):
        D = d_ref[...]
        jm = jnp.argmin(D, axis=1, keepdims=True).astype(jnp.int32)
        M1 = lane == jm
        M1f = M1.astype(jnp.float32)
        if coord_only:
            nbx = jnp.sum(M1f * px, axis=1, keepdims=True)
            nby = jnp.sum(M1f * py, axis=1, keepdims=True)
            nbz = jnp.sum(M1f * pz, axis=1, keepdims=True)
            g = jnp.concatenate(
                [nbx - cxc, nby - cyc, nbz - czc, nbx, nby, nbz], axis=1)
        else:
            g = _mm(M1f, F) - CC
        y = _mmd(g, W) + bb
        y_ref[0, r] = y
        d_ref[...] = jnp.where(M1, 1e30, D)
        return 0

    jax.lax.fori_loop(0, k, round_fn, 0)
    yall = jnp.reshape(y_ref[0], (k * S, Cout))
    ssum_ref[0] = jnp.sum(yall, axis=0, keepdims=True)
    ssq_ref[0] = jnp.sum(yall * yall, axis=0, keepdims=True)


def _knn_call(px, py, pz, cx, cy, cz, feats, wt, bb, k):
    B, N = px.shape
    S = cx.shape[1]
    Cin, Cout = wt.shape
    px = px[:, None, :]
    py = py[:, None, :]
    pz = pz[:, None, :]
    cx = cx[:, None, :]
    cy = cy[:, None, :]
    cz = cz[:, None, :]
    coord_only = feats is None
    body = functools.partial(_knn_body, k=k, coord_only=coord_only)
    row_specs = [pl.BlockSpec((1, 1, N), lambda b: (b, 0, 0))] * 3 + [
        pl.BlockSpec((1, 1, S), lambda b: (b, 0, 0))] * 3
    if coord_only:
        feats_args = ()
        feats_specs = []
    else:
        feats_args = (feats,)
        feats_specs = [pl.BlockSpec((1, N, Cin), lambda b: (b, 0, 0))]

    def body_wrap(*refs):
        if coord_only:
            (px_r, py_r, pz_r, cx_r, cy_r, cz_r, w_r, b_r,
             y_r, s_r, q_r, d_r) = refs
            body(px_r, py_r, pz_r, cx_r, cy_r, cz_r, None, w_r, b_r,
                 y_r, s_r, q_r, d_r)
        else:
            body(*refs)

    return pl.pallas_call(
        body_wrap,
        grid=(B,),
        in_specs=row_specs + feats_specs + [
            pl.BlockSpec((Cin, Cout), lambda b: (0, 0)),
            pl.BlockSpec((1, Cout), lambda b: (0, 0)),
        ],
        out_specs=[
            pl.BlockSpec((1, k, S, Cout), lambda b: (b, 0, 0, 0)),
            pl.BlockSpec((1, 1, Cout), lambda b: (b, 0, 0)),
            pl.BlockSpec((1, 1, Cout), lambda b: (b, 0, 0)),
        ],
        out_shape=[
            jax.ShapeDtypeStruct((B, k, S, Cout), jnp.float32),
            jax.ShapeDtypeStruct((B, 1, Cout), jnp.float32),
            jax.ShapeDtypeStruct((B, 1, Cout), jnp.float32),
        ],
        scratch_shapes=[pltpu.VMEM((S, N), jnp.float32)],
    )(px, py, pz, cx, cy, cz, *feats_args, wt, bb)


def _bn_conv_max_body(y_ref, ssum_ref, ssq_ref, g_ref, be_ref, w_ref, b2_ref,
                      out_ref, *, count):
    K, S, C = y_ref.shape[1:]
    C2 = w_ref.shape[1]
    nb = ssum_ref.shape[0]
    m = jnp.sum(jnp.reshape(ssum_ref[...], (nb, C)), axis=0,
                keepdims=True) / count
    ex2 = jnp.sum(jnp.reshape(ssq_ref[...], (nb, C)), axis=0,
                  keepdims=True) / count
    v = ex2 - m * m
    den = jnp.sqrt(v + 1e-5)
    yn = (y_ref[0] - m[None]) / den[None]
    h = jnp.maximum(g_ref[...][None] * yn + be_ref[...][None], 0.0)
    o = _mmd(jnp.reshape(h, (K * S, C)), w_ref[...]) + b2_ref[...]
    out_ref[0] = jnp.max(jnp.reshape(o, (K, S, C2)), axis=0)


def _pass2_call(y, ssum, ssq, g, be, wt, b2):
    B, K, S, C = y.shape
    C2 = wt.shape[1]
    body = functools.partial(_bn_conv_max_body, count=float(B * K * S))
    return pl.pallas_call(
        body,
        grid=(B,),
        in_specs=[
            pl.BlockSpec((1, K, S, C), lambda b: (b, 0, 0, 0)),
            pl.BlockSpec((B, 1, C), lambda b: (0, 0, 0)),
            pl.BlockSpec((B, 1, C), lambda b: (0, 0, 0)),
            pl.BlockSpec((1, C), lambda b: (0, 0)),
            pl.BlockSpec((1, C), lambda b: (0, 0)),
            pl.BlockSpec((C, C2), lambda b: (0, 0)),
            pl.BlockSpec((1, C2), lambda b: (0, 0)),
        ],
        out_specs=pl.BlockSpec((1, S, C2), lambda b: (b, 0, 0)),
        out_shape=jax.ShapeDtypeStruct((B, S, C2), jnp.float32),
    )(y, ssum, ssq, g, be, wt, b2)


def _tail_body(x_ref, w3a_ref, b3a_ref, g3_ref,
               be3_ref, w3b_ref, b3b_ref, fw1_ref, fb1_ref, fg1_ref, fbe1_ref,
               fw2_ref, fb2_ref, fg2_ref, fbe2_ref, fw3_ref, fb3_ref, out_ref,
               *, nb):
    B = nb
    S = x_ref.shape[0] // nb
    y = _mmd(x_ref[...], w3a_ref[...]) + b3a_ref[...]
    m = jnp.mean(y, axis=0, keepdims=True)
    v = jnp.mean((y - m) ** 2, axis=0, keepdims=True)
    yn = (y - m) / jnp.sqrt(v + 1e-5)
    h = jnp.maximum(g3_ref[...] * yn + be3_ref[...], 0.0)
    y2 = _mmd(h, w3b_ref[...]) + b3b_ref[...]
    gf = jnp.max(jnp.reshape(y2, (B, S, -1)), axis=1)

    def fc_bn_lrelu(x, wt, bv, gv, bev):
        hh = _mmd(x, wt) + bv
        mu = jnp.mean(hh, axis=0, keepdims=True)
        vv = jnp.mean((hh - mu) ** 2, axis=0, keepdims=True)
        nn = gv * (hh - mu) / jnp.sqrt(vv + 1e-5) + bev
        return jnp.where(nn >= 0, nn, 0.2 * nn)

    h1 = fc_bn_lrelu(gf, fw1_ref[...], fb1_ref[...], fg1_ref[...],
                     fbe1_ref[...])
    h2 = fc_bn_lrelu(h1, fw2_ref[...], fb2_ref[...], fg2_ref[...],
                     fbe2_ref[...])
    out_ref[...] = _mmd(h2, fw3_ref[...]) + fb3_ref[...]


def _tail_call(cx, cy, cz, l2, *weights):
    B, S = cx.shape
    C = l2.shape[2]
    coords = jnp.stack([cx, cy, cz], axis=2).reshape(B * S, 3)
    x = jnp.concatenate([coords, l2.reshape(B * S, C)], axis=1)
    body = functools.partial(_tail_body, nb=B)
    return pl.pallas_call(
        body,
        out_shape=jax.ShapeDtypeStruct((B, 1), jnp.float32),
    )(x, *weights)


def kernel(points, w1a, b1a, g1a, be1a, w1b, b1b, w2a, b2a, g2a, be2a, w2b,
           b2b, w3a, b3a, g3a, be3a, w3b, b3b, fw1, fb1, fg1, fbe1, fw2, fb2,
           fg2, fbe2, fw3, fb3):
    points = points.astype(jnp.float32)
    px = points[:, :, 0]
    py = points[:, :, 1]
    pz = points[:, :, 2]

    c1x, c1y, c1z = _fps_call(px, py, pz, 512)
    y1, s1, q1 = _knn_call(px, py, pz, c1x, c1y, c1z, None,
                           w1a.T, b1a[None], k=16)
    l1 = _pass2_call(y1, s1, q1, g1a[None], be1a[None], w1b.T, b1b[None])

    c2x, c2y, c2z = _fps_call(c1x, c1y, c1z, 128)
    feats2 = jnp.concatenate([jnp.stack([c1x, c1y, c1z], axis=2), l1], axis=2)
    y2, s2, q2 = _knn_call(c1x, c1y, c1z, c2x, c2y, c2z, feats2,
                           w2a.T, b2a[None], k=16)
    l2 = _pass2_call(y2, s2, q2, g2a[None], be2a[None], w2b.T, b2b[None])

    return _tail_call(c2x, c2y, c2z, l2,
                      w3a.T, b3a[None], g3a[None], be3a[None],
                      w3b.T, b3b[None],
                      fw1.T, fb1[None], fg1[None], fbe1[None],
                      fw2.T, fb2[None], fg2[None], fbe2[None],
                      fw3.T, fb3[None])
